# baseline (speedup 1.0000x reference)
import functools

import jax
import jax.numpy as jnp
from jax import lax
from jax.experimental import pallas as pl
from jax.experimental.pallas import tpu as pltpu
import jax.experimental.pallas.tpu_sc as plsc

F32 = jnp.float32
EPS = 1e-6

N = 50000
E = 400000
F = 128
B = 2

NTILES = 32
EBD = 2000
NBD = E // EBD
NCHUNK = 25
C = N // NCHUNK
TRASH = 48
CP = C + TRASH
ZR = CP // 16
EPT = E // 16
EB = 5000
NBLK = EPT // EB
NFULL = EB // 16
REM = EB - NFULL * 16
G = 128
HL = EB + G
WS = 400
NP = 50176

_mesh = plsc.VectorSubcoreMesh(core_axis_name="c", subcore_axis_name="s")


@functools.partial(
    pl.kernel,
    out_type=(
        jax.ShapeDtypeStruct((NTILES * NP,), F32),
        jax.ShapeDtypeStruct((NTILES * NP,), F32),
    ),
    mesh=_mesh,
    compiler_params=pltpu.CompilerParams(needs_layout_passes=False),
    scratch_types=[
        pltpu.VMEM((EBD,), jnp.int32),
        pltpu.VMEM((EBD,), jnp.int32),
        pltpu.VMEM((N,), F32),
        pltpu.VMEM((N,), F32),
    ],
)
def _deg(srcs, dsts, po, pi, src_v, dst_v, ho, hi):
    c = lax.axis_index("c")
    s = lax.axis_index("s")
    wid = c * 16 + s
    ones = jnp.full((16,), 1.0, F32)
    zeros = jnp.zeros((16,), F32)

    def zvec(i, _):
        ho[pl.ds(i * 16, 16)] = zeros
        hi[pl.ds(i * 16, 16)] = zeros
        return 0

    lax.fori_loop(0, N // 16, zvec, 0)

    def scan_block(blk):
        e0 = blk * EBD
        pltpu.sync_copy(srcs.at[pl.ds(e0, EBD)], src_v)
        pltpu.sync_copy(dsts.at[pl.ds(e0, EBD)], dst_v)

        def vbody(j, _):
            sv = src_v[pl.ds(j * 16, 16)]
            dv = dst_v[pl.ds(j * 16, 16)]
            plsc.addupdate_scatter(ho, [sv], ones)
            plsc.addupdate_scatter(hi, [dv], ones)
            return 0

        lax.fori_loop(0, EBD // 16, vbody, 0)

    def blkbody(i, _):
        scan_block(wid + NTILES * i)
        return 0

    lax.fori_loop(0, NBD // NTILES, blkbody, 0)

    if NBD % NTILES:
        @pl.when(wid < NBD % NTILES)
        def _():
            scan_block((NBD // NTILES) * NTILES + wid)

    pltpu.sync_copy(ho, po.at[pl.ds(wid * NP, N)])
    pltpu.sync_copy(hi, pi.at[pl.ds(wid * NP, N)])


KC = (NCHUNK + 1) // 2
AC = EPT + KC * 4 * G + 8


@functools.partial(
    pl.kernel,
    out_type=jax.ShapeDtypeStruct((B, N, F), F32),
    mesh=_mesh,
    compiler_params=pltpu.CompilerParams(needs_layout_passes=False),
    scratch_types=[
        pltpu.VMEM((EB,), jnp.int32),
        pltpu.VMEM((EB,), jnp.int32),
        pltpu.VMEM((AC,), jnp.int32),
        pltpu.VMEM((AC,), jnp.int32),
        pltpu.VMEM((G,), jnp.int32),
        pltpu.VMEM((G,), jnp.int32),
        pltpu.VMEM((G,), jnp.int32),
        pltpu.VMEM((G,), jnp.int32),
        pltpu.VMEM((G, F), F32),
        pltpu.VMEM((G, F), F32),
        pltpu.VMEM((32, F), F32),
        pltpu.VMEM_SHARED((CP, F), F32),
        pltpu.SMEM((32,), jnp.int32),
        pltpu.SemaphoreType.DMA,
        pltpu.SemaphoreType.DMA,
        pltpu.SemaphoreType.DMA,
        pltpu.SemaphoreType.DMA,
    ],
)
def _apply(g0, g1, srcs, dsts, U, src_v, dst_v, asrc, adst, gidx0, sidx0,
           gidx1, sidx1, grow0, grow1, zb, acc, smeta, gsem0, gsem1,
           ssem0, ssem1):
    c = lax.axis_index("c")
    s = lax.axis_index("s")
    iota = lax.iota(jnp.int32, 16)
    zeros = jnp.zeros((16,), F32)

    def zvec(i, _):
        zb[i // (F // 16), pl.ds((i % (F // 16)) * 16, 16)] = zeros
        return 0

    lax.fori_loop(0, 32 * (F // 16), zvec, 0)

    pads = (iota * 401 + s * 29) % N
    tvec = C + ((s * 16 + iota) % TRASH)
    los = [(2 * i + c) * C for i in range(KC)]

    def pop16(m):
        p = plsc.all_reduce_population_count(m)
        return lax.squeeze(lax.slice(p, (0,), (1,)), (0,))

    def load_block(bi):
        e0 = s * EPT + bi * EB
        pltpu.sync_copy(srcs.at[pl.ds(e0, EB)], src_v)
        pltpu.sync_copy(dsts.at[pl.ds(e0, EB)], dst_v)

    def masks(dv, valid_from):
        ms = []
        for i in range(KC):
            m = (dv >= los[i]) & (dv < los[i] + C)
            if valid_from:
                m = m & (iota >= valid_from)
            ms.append(m)
        return ms

    def p1_blk(bi, cnts):
        load_block(bi)

        def p1v(off, valid_from, cnts):
            dv = dst_v[pl.ds(off, 16)]
            ms = masks(dv, valid_from)
            return tuple(cnts[i] + pop16(ms[i]) for i in range(KC))

        def body(j, cnts):
            return p1v(j * 16, 0, cnts)

        cnts = lax.fori_loop(0, NFULL, body, cnts)
        if REM:
            cnts = p1v(EB - 16, 16 - REM, cnts)
        return cnts

    cnts = lax.fori_loop(0, NBLK, p1_blk, (jnp.int32(0),) * KC)

    nrs = [((cnts[i] + (2 * G - 1)) // (2 * G)) * 2 for i in range(KC)]
    seg = []
    off = jnp.int32(0)
    for i in range(KC):
        seg.append(off)
        smeta[i] = off
        smeta[16 + i] = nrs[i]
        off = off + nrs[i] * G + 2 * G

    def p2_blk(bi, ws):
        load_block(bi)

        def p2v(off, valid_from, ws):
            sv = src_v[pl.ds(off, 16)]
            dv = dst_v[pl.ds(off, 16)]
            ms = masks(dv, valid_from)
            out = []
            for i in range(KC):
                plsc.store_compressed(asrc.at[pl.ds(ws[i], 16)], sv,
                                      mask=ms[i])
                plsc.store_compressed(adst.at[pl.ds(ws[i], 16)], dv - los[i],
                                      mask=ms[i])
                out.append(ws[i] + pop16(ms[i]))
            return tuple(out)

        def body(j, ws):
            return p2v(j * 16, 0, ws)

        ws = lax.fori_loop(0, NFULL, body, ws)
        if REM:
            ws = p2v(EB - 16, 16 - REM, ws)
        return ws

    ws = lax.fori_loop(0, NBLK, p2_blk, tuple(seg))

    for i in range(KC):
        for jj in range(2 * G // 16):
            asrc[pl.ds(ws[i] + 16 * jj, 16)] = pads
            adst[pl.ds(ws[i] + 16 * jj, 16)] = tvec

    def stage(base, r, gi, si):
        def cpy(jj, _4):
            gi[pl.ds(jj * 16, 16)] = asrc[pl.ds(base + r * G + jj * 16, 16)]
            si[pl.ds(jj * 16, 16)] = adst[pl.ds(base + r * G + jj * 16, 16)]
            return 0

        lax.fori_loop(0, G // 16, cpy, 0)

    for b, gb in ((0, g0), (1, g1)):
        def chunk_body(ci, _):
            lo = (2 * ci + c) * C
            base = smeta[ci]
            nr = smeta[16 + ci]

            def zstripe(j, _2):
                pltpu.sync_copy(zb, acc.at[pl.ds(s * ZR + j * 32, 32)])
                return 0

            lax.fori_loop(0, ZR // 32, zstripe, 0)
            plsc.subcore_barrier()

            @pl.when(nr > 0)
            def _():
                stage(base, 0, gidx0, sidx0)
                pltpu.async_copy(gb.at[gidx0], grow0, gsem0)
                stage(base, 1, gidx1, sidx1)
                pltpu.async_copy(gb.at[gidx1], grow1, gsem1)

            def pair_body(r2, _3):
                r0 = 2 * r2

                pltpu.make_async_copy(gb.at[gidx0], grow0, gsem0).wait()
                pltpu.async_copy(grow0, acc.at[sidx0], ssem0, add=True)
                pltpu.make_async_copy(gb.at[gidx1], grow1, gsem1).wait()
                pltpu.async_copy(grow1, acc.at[sidx1], ssem1, add=True)

                @pl.when(r0 + 2 < nr)
                def _():
                    pltpu.make_async_copy(grow0, acc.at[sidx0], ssem0).wait()
                    stage(base, r0 + 2, gidx0, sidx0)
                    pltpu.async_copy(gb.at[gidx0], grow0, gsem0)
                    pltpu.make_async_copy(grow1, acc.at[sidx1], ssem1).wait()
                    stage(base, r0 + 3, gidx1, sidx1)
                    pltpu.async_copy(gb.at[gidx1], grow1, gsem1)

                @pl.when(r0 + 2 >= nr)
                def _():
                    pltpu.make_async_copy(grow0, acc.at[sidx0], ssem0).wait()
                    pltpu.make_async_copy(grow1, acc.at[sidx1], ssem1).wait()

                return 0

            lax.fori_loop(0, nr // 2, pair_body, 0)
            plsc.subcore_barrier()

            @pl.when((s < C // WS) & (2 * ci + c < NCHUNK))
            def _():
                pltpu.sync_copy(acc.at[pl.ds(s * WS, WS)],
                                U.at[b, pl.ds(lo + s * WS, WS)])

            plsc.subcore_barrier()
            return 0

        lax.fori_loop(0, KC, chunk_body, 0)


def _ln(h, scale, bias):
    mu = jnp.mean(h, axis=-1, keepdims=True)
    d = h - mu
    var = jnp.mean(d * d, axis=-1, keepdims=True)
    return d * lax.rsqrt(var + EPS) * scale + bias


_DN = (((0,), (0,)), ((), ()))


def _ab(po, pi):
    rd = 1024
    nb = NP // rd

    def body(po_ref, pi_ref, a_ref, b_ref):
        ones = jnp.ones((NTILES, 1), F32)
        do = lax.dot_general(po_ref[...], ones, _DN,
                             preferred_element_type=F32)
        di = lax.dot_general(pi_ref[...], ones, _DN,
                             preferred_element_type=F32)
        a_ref[...] = lax.rsqrt(jnp.maximum(do, 1.0))
        b_ref[...] = lax.rsqrt(jnp.maximum(di, 1.0))

    return pl.pallas_call(
        body,
        grid=(nb,),
        in_specs=[
            pl.BlockSpec((NTILES, rd), lambda i: (0, i)),
            pl.BlockSpec((NTILES, rd), lambda i: (0, i)),
        ],
        out_specs=[
            pl.BlockSpec((rd, 1), lambda i: (i, 0)),
            pl.BlockSpec((rd, 1), lambda i: (i, 0)),
        ],
        out_shape=[
            jax.ShapeDtypeStruct((NP, 1), F32),
            jax.ShapeDtypeStruct((NP, 1), F32),
        ],
    )(po, pi)


def _scale(x, a):
    rc = 2000

    def body(x_ref, a_ref, g_ref):
        g_ref[...] = a_ref[...][None] * x_ref[...]

    return pl.pallas_call(
        body,
        grid=(B, N // rc),
        in_specs=[
            pl.BlockSpec((1, rc, F), lambda bb, i: (bb, i, 0)),
            pl.BlockSpec((rc, 1), lambda bb, i: (i, 0)),
        ],
        out_specs=pl.BlockSpec((1, rc, F), lambda bb, i: (bb, i, 0)),
        out_shape=jax.ShapeDtypeStruct((B, N, F), F32),
    )(x, a)


def _combine1(U, a, b):
    rc = 2000

    def body(u_ref, a_ref, b_ref, t_ref, g_ref):
        t = -(b_ref[...][None] * u_ref[...])
        t_ref[...] = t
        g_ref[...] = a_ref[...][None] * t

    return pl.pallas_call(
        body,
        grid=(B, N // rc),
        in_specs=[
            pl.BlockSpec((1, rc, F), lambda bb, i: (bb, i, 0)),
            pl.BlockSpec((rc, 1), lambda bb, i: (i, 0)),
            pl.BlockSpec((rc, 1), lambda bb, i: (i, 0)),
        ],
        out_specs=[
            pl.BlockSpec((1, rc, F), lambda bb, i: (bb, i, 0)),
            pl.BlockSpec((1, rc, F), lambda bb, i: (bb, i, 0)),
        ],
        out_shape=[
            jax.ShapeDtypeStruct((B, N, F), F32),
            jax.ShapeDtypeStruct((B, N, F), F32),
        ],
    )(U, a, b)


def _combine2(U, Tm2, a, b):
    rc = 2000

    def body(u_ref, tm2_ref, a_ref, b_ref, t_ref, g_ref):
        t = -2.0 * (b_ref[...][None] * u_ref[...]) - tm2_ref[...]
        t_ref[...] = t
        g_ref[...] = a_ref[...][None] * t

    return pl.pallas_call(
        body,
        grid=(B, N // rc),
        in_specs=[
            pl.BlockSpec((1, rc, F), lambda bb, i: (bb, i, 0)),
            pl.BlockSpec((1, rc, F), lambda bb, i: (bb, i, 0)),
            pl.BlockSpec((rc, 1), lambda bb, i: (i, 0)),
            pl.BlockSpec((rc, 1), lambda bb, i: (i, 0)),
        ],
        out_specs=[
            pl.BlockSpec((1, rc, F), lambda bb, i: (bb, i, 0)),
            pl.BlockSpec((1, rc, F), lambda bb, i: (bb, i, 0)),
        ],
        out_shape=[
            jax.ShapeDtypeStruct((B, N, F), F32),
            jax.ShapeDtypeStruct((B, N, F), F32),
        ],
    )(U, Tm2, a, b)


def _e1(T0, T1, T2, U3, b, Wc, bc, s1, b1):
    rb = 2000

    def body(t0_ref, t1_ref, t2_ref, u3_ref, b_ref, wc_ref, bc_ref,
             s1_ref, b1_ref, h_ref):
        t1 = t1_ref[0]
        t3 = -2.0 * (b_ref[...] * u3_ref[0]) - t1
        h = bc_ref[...]
        h = h + jnp.dot(t0_ref[0], wc_ref[0], preferred_element_type=F32)
        h = h + jnp.dot(t1, wc_ref[1], preferred_element_type=F32)
        h = h + jnp.dot(t2_ref[0], wc_ref[2], preferred_element_type=F32)
        h = h + jnp.dot(t3, wc_ref[3], preferred_element_type=F32)
        h = jnp.maximum(h, 0.0)
        h_ref[...] = _ln(h, s1_ref[...], b1_ref[...])[None]

    return pl.pallas_call(
        body,
        grid=(B, N // rb),
        in_specs=[
            pl.BlockSpec((1, rb, F), lambda bb, i: (bb, i, 0)),
            pl.BlockSpec((1, rb, F), lambda bb, i: (bb, i, 0)),
            pl.BlockSpec((1, rb, F), lambda bb, i: (bb, i, 0)),
            pl.BlockSpec((1, rb, F), lambda bb, i: (bb, i, 0)),
            pl.BlockSpec((rb, 1), lambda bb, i: (i, 0)),
            pl.BlockSpec((4, F, F), lambda bb, i: (0, 0, 0)),
            pl.BlockSpec((1, F), lambda bb, i: (0, 0)),
            pl.BlockSpec((1, F), lambda bb, i: (0, 0)),
            pl.BlockSpec((1, F), lambda bb, i: (0, 0)),
        ],
        out_specs=pl.BlockSpec((1, rb, F), lambda bb, i: (bb, i, 0)),
        out_shape=jax.ShapeDtypeStruct((B, N, F), F32),
    )(T0, T1, T2, U3, b, Wc, bc, s1, b1)


def _e2(h4, Wp, bp, s2, b2):
    r4 = 1000
    n4 = B * (N // 4)

    def body(h_ref, wp_ref, bp_ref, s2_ref, b2_ref, o_ref):
        v = jnp.dot(h_ref[...], wp_ref[...], preferred_element_type=F32)
        v = jnp.maximum(v + bp_ref[...], 0.0)
        o_ref[...] = _ln(v, s2_ref[...], b2_ref[...])

    return pl.pallas_call(
        body,
        grid=(n4 // r4,),
        in_specs=[
            pl.BlockSpec((r4, 4 * F), lambda i: (i, 0)),
            pl.BlockSpec((4 * F, F), lambda i: (0, 0)),
            pl.BlockSpec((1, F), lambda i: (0, 0)),
            pl.BlockSpec((1, F), lambda i: (0, 0)),
            pl.BlockSpec((1, F), lambda i: (0, 0)),
        ],
        out_specs=pl.BlockSpec((r4, F), lambda i: (i, 0)),
        out_shape=jax.ShapeDtypeStruct((n4, F), F32),
    )(h4, Wp, bp, s2, b2)


def kernel(inputs, edge_index, W_cheb, b_cheb, ln1_scale, ln1_bias,
           W_pseudo, b_pseudo, ln2_scale, ln2_bias):
    assert inputs.shape == (B, N, F) and edge_index.shape == (2, E)
    srcs = edge_index[0]
    dsts = edge_index[1]
    po, pi = _deg(srcs, dsts)
    po = po.reshape(NTILES, NP)
    pi = pi.reshape(NTILES, NP)
    a, b = _ab(po, pi)
    a = a[:N]
    b = b[:N]
    g = _scale(inputs, a)
    T0 = inputs
    U1 = _apply(g[0], g[1], srcs, dsts)
    T1, g = _combine1(U1, a, b)
    U2 = _apply(g[0], g[1], srcs, dsts)
    T2, g = _combine2(U2, T0, a, b)
    U3 = _apply(g[0], g[1], srcs, dsts)
    h = _e1(T0, T1, T2, U3, b, W_cheb.reshape(4, F, F),
            b_cheb.reshape(1, F), ln1_scale.reshape(1, F),
            ln1_bias.reshape(1, F))
    h4 = h.reshape(B * (N // 4), 4 * F)
    out = _e2(h4, W_pseudo, b_pseudo.reshape(1, F),
              ln2_scale.reshape(1, F), ln2_bias.reshape(1, F))
    return out.reshape(B, N // 4, F)


# speedup vs baseline: 1.1940x; 1.1940x over previous
import functools

import jax
import jax.numpy as jnp
from jax import lax
from jax.experimental import pallas as pl
from jax.experimental.pallas import tpu as pltpu
import jax.experimental.pallas.tpu_sc as plsc

F32 = jnp.float32
EPS = 1e-6

N = 50000
E = 400000
F = 128
B = 2

NTILES = 32
EBD = 2000
NBD = E // EBD
NCHUNK = 25
C = N // NCHUNK
TRASH = 48
CP = C + TRASH
ZR = CP // 16
EPT = E // 16
EB = 5000
NBLK = EPT // EB
NFULL = EB // 16
REM = EB - NFULL * 16
G = 128
HL = EB + G
WS = 400
NP = 50176

_mesh = plsc.VectorSubcoreMesh(core_axis_name="c", subcore_axis_name="s")


@functools.partial(
    pl.kernel,
    out_type=(
        jax.ShapeDtypeStruct((NTILES * NP,), F32),
        jax.ShapeDtypeStruct((NTILES * NP,), F32),
    ),
    mesh=_mesh,
    compiler_params=pltpu.CompilerParams(needs_layout_passes=False),
    scratch_types=[
        pltpu.VMEM((EBD,), jnp.int32),
        pltpu.VMEM((EBD,), jnp.int32),
        pltpu.VMEM((N,), F32),
        pltpu.VMEM((N,), F32),
    ],
)
def _deg(srcs, dsts, po, pi, src_v, dst_v, ho, hi):
    c = lax.axis_index("c")
    s = lax.axis_index("s")
    wid = c * 16 + s
    ones = jnp.full((16,), 1.0, F32)
    zeros = jnp.zeros((16,), F32)

    def zvec(i, _):
        ho[pl.ds(i * 16, 16)] = zeros
        hi[pl.ds(i * 16, 16)] = zeros
        return 0

    lax.fori_loop(0, N // 16, zvec, 0)

    def scan_block(blk):
        e0 = blk * EBD
        pltpu.sync_copy(srcs.at[pl.ds(e0, EBD)], src_v)
        pltpu.sync_copy(dsts.at[pl.ds(e0, EBD)], dst_v)

        def vbody(j, _):
            sv = src_v[pl.ds(j * 16, 16)]
            dv = dst_v[pl.ds(j * 16, 16)]
            plsc.addupdate_scatter(ho, [sv], ones)
            plsc.addupdate_scatter(hi, [dv], ones)
            return 0

        lax.fori_loop(0, EBD // 16, vbody, 0)

    def blkbody(i, _):
        scan_block(wid + NTILES * i)
        return 0

    lax.fori_loop(0, NBD // NTILES, blkbody, 0)

    if NBD % NTILES:
        @pl.when(wid < NBD % NTILES)
        def _():
            scan_block((NBD // NTILES) * NTILES + wid)

    pltpu.sync_copy(ho, po.at[pl.ds(wid * NP, N)])
    pltpu.sync_copy(hi, pi.at[pl.ds(wid * NP, N)])


KC = (NCHUNK + 1) // 2
AC = EPT + KC * 4 * G + 8


@functools.partial(
    pl.kernel,
    out_type=jax.ShapeDtypeStruct((B, N, F), F32),
    mesh=_mesh,
    compiler_params=pltpu.CompilerParams(needs_layout_passes=False),
    scratch_types=[
        pltpu.VMEM((EB,), jnp.int32),
        pltpu.VMEM((EB,), jnp.int32),
        pltpu.VMEM((AC,), jnp.int32),
        pltpu.VMEM((AC,), jnp.int32),
        pltpu.VMEM((G,), jnp.int32),
        pltpu.VMEM((G,), jnp.int32),
        pltpu.VMEM((G,), jnp.int32),
        pltpu.VMEM((G,), jnp.int32),
        pltpu.VMEM((G, F), F32),
        pltpu.VMEM((G, F), F32),
        pltpu.VMEM((32, F), F32),
        pltpu.VMEM_SHARED((CP, F), F32),
        pltpu.SMEM((32,), jnp.int32),
        pltpu.SemaphoreType.DMA,
        pltpu.SemaphoreType.DMA,
        pltpu.SemaphoreType.DMA,
        pltpu.SemaphoreType.DMA,
    ],
)
def _apply(g0, g1, srcs, dsts, U, src_v, dst_v, asrc, adst, gidx0, sidx0,
           gidx1, sidx1, grow0, grow1, zb, acc, smeta, gsem0, gsem1,
           ssem0, ssem1):
    c = lax.axis_index("c")
    s = lax.axis_index("s")
    iota = lax.iota(jnp.int32, 16)
    zeros = jnp.zeros((16,), F32)

    def zvec(i, _):
        zb[i // (F // 16), pl.ds((i % (F // 16)) * 16, 16)] = zeros
        return 0

    lax.fori_loop(0, 32 * (F // 16), zvec, 0)

    pads = (iota * 401 + s * 29) % N
    tvec = C + ((s * 16 + iota) % TRASH)
    los = [(2 * i + c) * C for i in range(KC)]

    def pop16(m):
        p = plsc.all_reduce_population_count(m)
        return lax.squeeze(lax.slice(p, (0,), (1,)), (0,))

    def load_block(bi):
        e0 = s * EPT + bi * EB
        pltpu.sync_copy(srcs.at[pl.ds(e0, EB)], src_v)
        pltpu.sync_copy(dsts.at[pl.ds(e0, EB)], dst_v)

    def masks(dv, valid_from):
        ms = []
        for i in range(KC):
            m = (dv >= los[i]) & (dv < los[i] + C)
            if valid_from:
                m = m & (iota >= valid_from)
            ms.append(m)
        return ms

    def p1_blk(bi, cnts):
        load_block(bi)

        def p1v(off, valid_from, cnts):
            dv = dst_v[pl.ds(off, 16)]
            ms = masks(dv, valid_from)
            return tuple(cnts[i] + pop16(ms[i]) for i in range(KC))

        def body(j, cnts):
            return p1v(j * 16, 0, cnts)

        cnts = lax.fori_loop(0, NFULL, body, cnts)
        if REM:
            cnts = p1v(EB - 16, 16 - REM, cnts)
        return cnts

    cnts = lax.fori_loop(0, NBLK, p1_blk, (jnp.int32(0),) * KC)

    nrs = [((cnts[i] + (2 * G - 1)) // (2 * G)) * 2 for i in range(KC)]
    seg = []
    off = jnp.int32(0)
    for i in range(KC):
        seg.append(off)
        smeta[i] = off
        smeta[16 + i] = nrs[i]
        off = off + nrs[i] * G + 2 * G

    def p2_blk(bi, ws):
        load_block(bi)

        def p2v(off, valid_from, ws):
            sv = src_v[pl.ds(off, 16)]
            dv = dst_v[pl.ds(off, 16)]
            ms = masks(dv, valid_from)
            out = []
            for i in range(KC):
                plsc.store_compressed(asrc.at[pl.ds(ws[i], 16)], sv,
                                      mask=ms[i])
                plsc.store_compressed(adst.at[pl.ds(ws[i], 16)], dv - los[i],
                                      mask=ms[i])
                out.append(ws[i] + pop16(ms[i]))
            return tuple(out)

        def body(j, ws):
            return p2v(j * 16, 0, ws)

        ws = lax.fori_loop(0, NFULL, body, ws)
        if REM:
            ws = p2v(EB - 16, 16 - REM, ws)
        return ws

    ws = lax.fori_loop(0, NBLK, p2_blk, tuple(seg))

    for i in range(KC):
        for jj in range(2 * G // 16):
            asrc[pl.ds(ws[i] + 16 * jj, 16)] = pads
            adst[pl.ds(ws[i] + 16 * jj, 16)] = tvec

    def stage(base, r, gi, si):
        def cpy(jj, _4):
            gi[pl.ds(jj * 16, 16)] = asrc[pl.ds(base + r * G + jj * 16, 16)]
            si[pl.ds(jj * 16, 16)] = adst[pl.ds(base + r * G + jj * 16, 16)]
            return 0

        lax.fori_loop(0, G // 16, cpy, 0)

    for b, gb in ((0, g0), (1, g1)):
        def chunk_body(ci, _):
            lo = (2 * ci + c) * C
            base = smeta[ci]
            nr = smeta[16 + ci]

            def zstripe(j, _2):
                pltpu.sync_copy(zb, acc.at[pl.ds(s * ZR + j * 32, 32)])
                return 0

            lax.fori_loop(0, ZR // 32, zstripe, 0)
            plsc.subcore_barrier()

            @pl.when(nr > 0)
            def _():
                stage(base, 0, gidx0, sidx0)
                pltpu.async_copy(gb.at[gidx0], grow0, gsem0)

            def pair_body(r2, _3):
                r0 = 2 * r2
                r1 = r0 + 1

                @pl.when(r1 < nr)
                def _():
                    stage(base, r1, gidx1, sidx1)
                    pltpu.async_copy(gb.at[gidx1], grow1, gsem1)

                pltpu.make_async_copy(gb.at[gidx0], grow0, gsem0).wait()
                pltpu.sync_copy(grow0, acc.at[sidx0], add=True)

                @pl.when(r0 + 2 < nr)
                def _():
                    stage(base, r0 + 2, gidx0, sidx0)
                    pltpu.async_copy(gb.at[gidx0], grow0, gsem0)

                @pl.when(r1 < nr)
                def _():
                    pltpu.make_async_copy(gb.at[gidx1], grow1, gsem1).wait()
                    pltpu.sync_copy(grow1, acc.at[sidx1], add=True)

                return 0

            lax.fori_loop(0, (nr + 1) // 2, pair_body, 0)
            plsc.subcore_barrier()

            @pl.when((s < C // WS) & (2 * ci + c < NCHUNK))
            def _():
                pltpu.sync_copy(acc.at[pl.ds(s * WS, WS)],
                                U.at[b, pl.ds(lo + s * WS, WS)])

            plsc.subcore_barrier()
            return 0

        lax.fori_loop(0, KC, chunk_body, 0)


def _ln(h, scale, bias):
    mu = jnp.mean(h, axis=-1, keepdims=True)
    d = h - mu
    var = jnp.mean(d * d, axis=-1, keepdims=True)
    return d * lax.rsqrt(var + EPS) * scale + bias


_DN = (((0,), (0,)), ((), ()))


def _ab(po, pi):
    rd = 1024
    nb = NP // rd

    def body(po_ref, pi_ref, a_ref, b_ref, ab_ref):
        ones = jnp.ones((NTILES, 1), F32)
        do = lax.dot_general(po_ref[...], ones, _DN,
                             preferred_element_type=F32)
        di = lax.dot_general(pi_ref[...], ones, _DN,
                             preferred_element_type=F32)
        a = lax.rsqrt(jnp.maximum(do, 1.0))
        b = lax.rsqrt(jnp.maximum(di, 1.0))
        a_ref[...] = a
        b_ref[...] = b
        ab_ref[...] = a * b

    return pl.pallas_call(
        body,
        grid=(nb,),
        in_specs=[
            pl.BlockSpec((NTILES, rd), lambda i: (0, i)),
            pl.BlockSpec((NTILES, rd), lambda i: (0, i)),
        ],
        out_specs=[
            pl.BlockSpec((rd, 1), lambda i: (i, 0)),
            pl.BlockSpec((rd, 1), lambda i: (i, 0)),
            pl.BlockSpec((rd, 1), lambda i: (i, 0)),
        ],
        out_shape=[
            jax.ShapeDtypeStruct((NP, 1), F32),
            jax.ShapeDtypeStruct((NP, 1), F32),
            jax.ShapeDtypeStruct((NP, 1), F32),
        ],
    )(po, pi)


def _scale(x, a):
    rc = 2000

    def body(x_ref, a_ref, g_ref):
        g_ref[...] = a_ref[...][None] * x_ref[...]

    return pl.pallas_call(
        body,
        grid=(B, N // rc),
        in_specs=[
            pl.BlockSpec((1, rc, F), lambda bb, i: (bb, i, 0)),
            pl.BlockSpec((rc, 1), lambda bb, i: (i, 0)),
        ],
        out_specs=pl.BlockSpec((1, rc, F), lambda bb, i: (bb, i, 0)),
        out_shape=jax.ShapeDtypeStruct((B, N, F), F32),
    )(x, a)


def _combine2(U, Ym2, nab2):
    rc = 2000

    def body(u_ref, ym2_ref, nab2_ref, y_ref):
        y_ref[...] = nab2_ref[...][None] * u_ref[...] - ym2_ref[...]

    return pl.pallas_call(
        body,
        grid=(B, N // rc),
        in_specs=[
            pl.BlockSpec((1, rc, F), lambda bb, i: (bb, i, 0)),
            pl.BlockSpec((1, rc, F), lambda bb, i: (bb, i, 0)),
            pl.BlockSpec((rc, 1), lambda bb, i: (i, 0)),
        ],
        out_specs=pl.BlockSpec((1, rc, F), lambda bb, i: (bb, i, 0)),
        out_shape=jax.ShapeDtypeStruct((B, N, F), F32),
    )(U, Ym2, nab2)


def _epi(x2, U1, U2, U3, b2, Wc, bc, s1, b1, Wp, bp, s2, b2ln):
    rb = 4000
    nb = B * N // rb

    def body(x_ref, u1_ref, u2_ref, u3_ref, b_ref, wc_ref, bc_ref,
             s1_ref, b1_ref, wp_ref, bp_ref, s2_ref, b2_ref, o_ref):
        bcol = b_ref[...]
        t0 = x_ref[...]
        t1 = -(bcol * u1_ref[...])
        t2 = -2.0 * (bcol * u2_ref[...]) - t0
        t3 = -2.0 * (bcol * u3_ref[...]) - t1
        h = bc_ref[...]
        h = h + jnp.dot(t0, wc_ref[0], preferred_element_type=F32)
        h = h + jnp.dot(t1, wc_ref[1], preferred_element_type=F32)
        h = h + jnp.dot(t2, wc_ref[2], preferred_element_type=F32)
        h = h + jnp.dot(t3, wc_ref[3], preferred_element_type=F32)
        h = jnp.maximum(h, 0.0)
        h = _ln(h, s1_ref[...], b1_ref[...])
        hr = h.reshape(rb // 4, 4, F)
        v = bp_ref[...]
        for j in range(4):
            v = v + jnp.dot(hr[:, j, :], wp_ref[j],
                            preferred_element_type=F32)
        v = jnp.maximum(v, 0.0)
        o_ref[...] = _ln(v, s2_ref[...], b2_ref[...])

    cspec = pl.BlockSpec((1, F), lambda i: (0, 0))
    return pl.pallas_call(
        body,
        grid=(nb,),
        in_specs=[
            pl.BlockSpec((rb, F), lambda i: (i, 0)),
            pl.BlockSpec((rb, F), lambda i: (i, 0)),
            pl.BlockSpec((rb, F), lambda i: (i, 0)),
            pl.BlockSpec((rb, F), lambda i: (i, 0)),
            pl.BlockSpec((rb, 1), lambda i: (i, 0)),
            pl.BlockSpec((4, F, F), lambda i: (0, 0, 0)),
            cspec,
            cspec,
            cspec,
            pl.BlockSpec((4, F, F), lambda i: (0, 0, 0)),
            cspec,
            cspec,
            cspec,
        ],
        out_specs=pl.BlockSpec((rb // 4, F), lambda i: (i, 0)),
        out_shape=jax.ShapeDtypeStruct((B * N // 4, F), F32),
    )(x2, U1, U2, U3, b2, Wc, bc, s1, b1, Wp, bp, s2, b2ln)


def kernel(inputs, edge_index, W_cheb, b_cheb, ln1_scale, ln1_bias,
           W_pseudo, b_pseudo, ln2_scale, ln2_bias):
    assert inputs.shape == (B, N, F) and edge_index.shape == (2, E)
    srcs = edge_index[0]
    dsts = edge_index[1]
    po, pi = _deg(srcs, dsts)
    po = po.reshape(NTILES, NP)
    pi = pi.reshape(NTILES, NP)
    a, b, ab = _ab(po, pi)
    a = a[:N]
    b = b[:N]
    nab = -ab[:N]
    nab2 = 2.0 * nab

    y0 = _scale(inputs, a)
    U1 = _apply(y0[0], y0[1], srcs, dsts)
    y1 = _scale(U1, nab)
    U2 = _apply(y1[0], y1[1], srcs, dsts)
    y2 = _combine2(U2, y0, nab2)
    U3 = _apply(y2[0], y2[1], srcs, dsts)

    b2 = jnp.concatenate([b, b], axis=0)
    out = _epi(inputs.reshape(B * N, F), U1.reshape(B * N, F),
               U2.reshape(B * N, F), U3.reshape(B * N, F), b2,
               W_cheb.reshape(4, F, F), b_cheb.reshape(1, F),
               ln1_scale.reshape(1, F), ln1_bias.reshape(1, F),
               W_pseudo.reshape(4, F, F), b_pseudo.reshape(1, F),
               ln2_scale.reshape(1, F), ln2_bias.reshape(1, F))
    return out.reshape(B, N // 4, F)
